# Initial kernel scaffold; baseline (speedup 1.0000x reference)
import jax
import jax.numpy as jnp
from jax.experimental import pallas as pl


def kernel(params, x, edge_index, batch):
    raise NotImplementedError("write your pallas kernel here")


# speedup vs baseline: 2.0726x; 2.0726x over previous
import functools

import jax
import jax.numpy as jnp
import numpy as np
from jax.experimental import pallas as pl
from jax.experimental.pallas import tpu as pltpu

N_NODES_C = 50000
N_GRAPHS_C = 200
HIDDEN_C = 16
MAX_ELEM_C = 500


def _final_matmul_body(pooled_ref, w_ref, b_ref, out_ref):
    out_ref[...] = (
        jnp.dot(pooled_ref[...], w_ref[...],
                preferred_element_type=jnp.float32,
                precision=jax.lax.Precision.HIGHEST)
        + b_ref[...]
    )


def _final_matmul(pooled, w, b):
    g, k = pooled.shape
    return pl.pallas_call(
        _final_matmul_body,
        out_shape=jax.ShapeDtypeStruct((g, k), jnp.float32),
    )(pooled, w, b[None, :])


def kernel(params, x, edge_index, batch):
    p = params
    N = x.shape[0]
    G = N_GRAPHS_C
    H = HIDDEN_C
    ME = MAX_ELEM_C
    t = p['t']

    c0 = jnp.repeat(jnp.arange(3), 3)
    c1 = jnp.tile(jnp.arange(3), 3)
    V = jnp.concatenate([p['emb'][c0], c1[:, None].astype(jnp.float32)], axis=1)
    cls = x[:, 0] * 3 + x[:, 1]
    src, dst = edge_index[0], edge_index[1]

    hist = jnp.zeros((N, 9), jnp.float32).at[dst, cls[src]].add(1.0)
    E1 = jnp.exp(V * t)
    denom1 = hist @ E1
    num1 = hist @ (E1 * V)
    aggr1 = num1 / (denom1 + 1e-16)
    xf = V[cls]
    h1 = jax.nn.relu(aggr1 @ p['Wl1'] + p['bl1'] + xf @ p['Wr1'])

    E2 = jnp.exp(h1 * t)
    Q = jnp.concatenate([E2 * h1, E2], axis=1)
    acc = jnp.zeros((N, 2 * H), jnp.float32).at[dst].add(Q[src])
    aggr2 = acc[:, :H] / (acc[:, H:] + 1e-16)
    h2 = jax.nn.relu(aggr2 @ p['Wl2'] + p['bl2'] + h1 @ p['Wr2'])

    counts = jnp.bincount(batch, length=G)
    starts = jnp.concatenate([jnp.zeros((1,), counts.dtype), jnp.cumsum(counts)[:-1]])
    L = jnp.max(counts)
    pidx = jnp.arange(ME)[None, :]
    gidx = starts[:, None] + pidx
    mask = pidx < counts[:, None]
    dense = jnp.where(mask[:, :, None],
                      h2[jnp.minimum(gidx, N - 1)], 0.0)

    mlp_out = dense.reshape(G, ME * H) @ p['Wmlp'] + p['bmlp']

    Wi, Wh, bi, bh = p['Wi'], p['Wh'], p['bi'], p['bh']

    def step(h, inp):
        tt, xt = inp
        gi = xt @ Wi + bi
        gh = h @ Wh + bh
        ir, iz, inn = jnp.split(gi, 3, axis=1)
        hr, hz, hn = jnp.split(gh, 3, axis=1)
        r = jax.nn.sigmoid(ir + hr)
        z = jax.nn.sigmoid(iz + hz)
        n = jnp.tanh(inn + r * hn)
        h_new = (1.0 - z) * n + z * h
        return jnp.where(tt < L, h_new, h), None

    h0 = jnp.zeros((G, H), jnp.float32)
    ts = jnp.arange(ME)
    gru_out, _ = jax.lax.scan(step, h0, (ts, jnp.swapaxes(dense, 0, 1)))

    def mab(pre, Qm, K, m):
        d = Qm.shape[-1]
        Qp = Qm @ p[pre + 'Wq'] + p[pre + 'bq']
        Kp = K @ p[pre + 'Wk'] + p[pre + 'bk']
        Vp = K @ p[pre + 'Wv'] + p[pre + 'bv']
        scores = jnp.einsum('bqd,bkd->bqk', Qp, Kp) / np.sqrt(d)
        scores = jnp.where(m[:, None, :], scores, -1e30)
        A = jax.nn.softmax(scores, axis=-1)
        out = Qp + jnp.einsum('bqk,bkd->bqd', A, Vp)
        return out + jax.nn.relu(out @ p[pre + 'Wo'] + p[pre + 'bo'])

    z = mab('enc_', dense, dense, mask)
    q = jnp.tile(p['S'], (G, 1, 1))
    kv = jax.nn.relu(z @ p['pma_lin_W'] + p['pma_lin_b'])
    st = mab('pma_', q, kv, mask)
    st_out = jnp.nan_to_num(st.reshape(G, H))

    pooled = jnp.concatenate([mlp_out, gru_out, st_out], axis=1)
    return _final_matmul(pooled, p['Wfin'], p['bfin'])


# speedup vs baseline: 33.6545x; 16.2379x over previous
import functools

import jax
import jax.numpy as jnp
import numpy as np
from jax import lax
from jax.experimental import pallas as pl
from jax.experimental.pallas import tpu as pltpu
from jax.experimental.pallas import tpu_sc as plsc

N_NODES_C = 50000
N_GRAPHS_C = 200
HIDDEN_C = 16
MAX_ELEM_C = 500

_NC, _NS = 2, 16
_NW = _NC * _NS
_CH = 1024
_ACC_PER_TILE = 3136
_ACC_ROWS = _ACC_PER_TILE * _NS


def _sc_edge_aggregate(src2d, dst2d, table3, split_features):
    e_pad = src2d.shape[0] * 128
    ntiles = _NS if split_features else _NW
    chunks = e_pad // (ntiles * _CH)
    rows_per_tile = chunks * (_CH // 128)
    mesh = plsc.VectorSubcoreMesh(core_axis_name="c", subcore_axis_name="s")

    @functools.partial(
        pl.kernel,
        out_type=jax.ShapeDtypeStruct((_NC, _ACC_ROWS, 16), jnp.float32),
        mesh=mesh,
        scratch_types=[
            pltpu.VMEM((8, 128), jnp.int32),
            pltpu.VMEM((8, 128), jnp.int32),
            pltpu.VMEM((_CH, 16), jnp.float32),
            pltpu.VMEM((784, 16), jnp.float32),
            pltpu.VMEM_SHARED((_ACC_ROWS, 16), jnp.float32),
            pltpu.SemaphoreType.DMA,
        ],
        compiler_params=pltpu.CompilerParams(use_tc_tiling_on_sc=False),
    )
    def k(src_hbm, dst_hbm, table_hbm, out_hbm, sidx, didx, rows, zbuf, acc, sem):
        ci = lax.axis_index("c")
        si = lax.axis_index("s")
        tid = si if split_features else si * _NC + ci
        tbl = table_hbm.at[ci] if split_features else table_hbm.at[0]

        @pl.loop(0, 784)
        def _(i):
            zbuf[i, :] = jnp.zeros((16,), jnp.float32)

        for q in range(4):
            pltpu.sync_copy(zbuf, acc.at[pl.ds(si * _ACC_PER_TILE + q * 784, 784)])
        plsc.subcore_barrier()

        @pl.loop(0, chunks)
        def _(c):
            row_base = tid * rows_per_tile + c * 8
            pltpu.sync_copy(src_hbm.at[pl.ds(row_base, 8)], sidx)
            pltpu.sync_copy(dst_hbm.at[pl.ds(row_base, 8)], didx)
            cps = [
                pltpu.async_copy(
                    tbl.at[sidx.at[j]], rows.at[pl.ds(j * 128, 128)], sem)
                for j in range(8)
            ]
            for cp in cps:
                cp.wait()
            for j in range(8):
                pltpu.sync_copy(
                    rows.at[pl.ds(j * 128, 128)], acc.at[didx.at[j]], add=True)

        plsc.subcore_barrier()

        pltpu.sync_copy(
            acc.at[pl.ds(si * _ACC_PER_TILE, _ACC_PER_TILE)],
            out_hbm.at[ci].at[pl.ds(si * _ACC_PER_TILE, _ACC_PER_TILE)])

    return k(src2d, dst2d, table3)


def _edge_aggregate(src, dst, table):
    e = src.shape[0]
    K = table.shape[1]
    unit = _NW * _CH
    e_pad = ((e + unit - 1) // unit) * unit
    src2d = jnp.pad(src, (0, e_pad - e)).reshape(-1, 128)
    dst2d = jnp.pad(dst, (0, e_pad - e),
                    constant_values=N_NODES_C).reshape(-1, 128)
    if K == 16:
        parts = _sc_edge_aggregate(src2d, dst2d, table[None], False)
        return parts[0, :N_NODES_C] + parts[1, :N_NODES_C]
    table3 = jnp.stack([table[:, :16], table[:, 16:]])
    parts = _sc_edge_aggregate(src2d, dst2d, table3, True)
    return jnp.concatenate([parts[0, :N_NODES_C], parts[1, :N_NODES_C]], axis=1)


def _final_matmul_body(pooled_ref, w_ref, b_ref, out_ref):
    out_ref[...] = (
        jnp.dot(pooled_ref[...], w_ref[...],
                preferred_element_type=jnp.float32,
                precision=jax.lax.Precision.HIGHEST)
        + b_ref[...]
    )


def _final_matmul(pooled, w, b):
    g, k = pooled.shape
    return pl.pallas_call(
        _final_matmul_body,
        out_shape=jax.ShapeDtypeStruct((g, k), jnp.float32),
    )(pooled, w, b[None, :])


def kernel(params, x, edge_index, batch):
    p = params
    N = x.shape[0]
    G = N_GRAPHS_C
    H = HIDDEN_C
    ME = MAX_ELEM_C
    t = p['t']

    c0 = jnp.repeat(jnp.arange(3), 3)
    c1 = jnp.tile(jnp.arange(3), 3)
    V = jnp.concatenate([p['emb'][c0], c1[:, None].astype(jnp.float32)], axis=1)
    cls = x[:, 0] * 3 + x[:, 1]
    src, dst = edge_index[0], edge_index[1]

    onehot = (cls[:, None] == jnp.arange(16)[None, :]).astype(jnp.float32)
    hist = _edge_aggregate(src, dst, onehot)[:, :9]
    E1 = jnp.exp(V * t)
    denom1 = hist @ E1
    num1 = hist @ (E1 * V)
    aggr1 = num1 / (denom1 + 1e-16)
    xf = V[cls]
    h1 = jax.nn.relu(aggr1 @ p['Wl1'] + p['bl1'] + xf @ p['Wr1'])

    E2 = jnp.exp(h1 * t)
    Q = jnp.concatenate([E2 * h1, E2], axis=1)
    acc = _edge_aggregate(src, dst, Q)
    aggr2 = acc[:, :H] / (acc[:, H:] + 1e-16)
    h2 = jax.nn.relu(aggr2 @ p['Wl2'] + p['bl2'] + h1 @ p['Wr2'])

    counts = jnp.bincount(batch, length=G)
    starts = jnp.concatenate([jnp.zeros((1,), counts.dtype), jnp.cumsum(counts)[:-1]])
    L = jnp.max(counts)
    pidx = jnp.arange(ME)[None, :]
    gidx = starts[:, None] + pidx
    mask = pidx < counts[:, None]
    dense = jnp.where(mask[:, :, None],
                      h2[jnp.minimum(gidx, N - 1)], 0.0)

    mlp_out = dense.reshape(G, ME * H) @ p['Wmlp'] + p['bmlp']

    Wi, Wh, bi, bh = p['Wi'], p['Wh'], p['bi'], p['bh']

    def step(h, inp):
        tt, xt = inp
        gi = xt @ Wi + bi
        gh = h @ Wh + bh
        ir, iz, inn = jnp.split(gi, 3, axis=1)
        hr, hz, hn = jnp.split(gh, 3, axis=1)
        r = jax.nn.sigmoid(ir + hr)
        z = jax.nn.sigmoid(iz + hz)
        n = jnp.tanh(inn + r * hn)
        h_new = (1.0 - z) * n + z * h
        return jnp.where(tt < L, h_new, h), None

    h0 = jnp.zeros((G, H), jnp.float32)
    ts = jnp.arange(ME)
    gru_out, _ = jax.lax.scan(step, h0, (ts, jnp.swapaxes(dense, 0, 1)))

    def mab(pre, Qm, K, m):
        d = Qm.shape[-1]
        Qp = Qm @ p[pre + 'Wq'] + p[pre + 'bq']
        Kp = K @ p[pre + 'Wk'] + p[pre + 'bk']
        Vp = K @ p[pre + 'Wv'] + p[pre + 'bv']
        scores = jnp.einsum('bqd,bkd->bqk', Qp, Kp) / np.sqrt(d)
        scores = jnp.where(m[:, None, :], scores, -1e30)
        A = jax.nn.softmax(scores, axis=-1)
        out = Qp + jnp.einsum('bqk,bkd->bqd', A, Vp)
        return out + jax.nn.relu(out @ p[pre + 'Wo'] + p[pre + 'bo'])

    z = mab('enc_', dense, dense, mask)
    q = jnp.tile(p['S'], (G, 1, 1))
    kv = jax.nn.relu(z @ p['pma_lin_W'] + p['pma_lin_b'])
    st = mab('pma_', q, kv, mask)
    st_out = jnp.nan_to_num(st.reshape(G, H))

    pooled = jnp.concatenate([mlp_out, gru_out, st_out], axis=1)
    return _final_matmul(pooled, p['Wfin'], p['bfin'])
